# baseline (speedup 1.0000x reference)
import functools

import jax
import jax.numpy as jnp
from jax import lax
from jax.experimental import pallas as pl
from jax.experimental.pallas import tpu as pltpu
from jax.experimental.pallas import tpu_sc as plsc

EMBED = 64
BATCH = 16384
FIELDS = 26
LANES = 16

NC, NS = 2, 16
NW = NC * NS
ROWS_W = BATCH // NW
BLK = 32
NBLK = ROWS_W // BLK
CHUNK = BLK * FIELDS

_mesh = plsc.VectorSubcoreMesh(
    core_axis_name="c", subcore_axis_name="s", num_cores=NC, num_subcores=NS
)


@functools.partial(
    pl.kernel,
    mesh=_mesh,
    out_type=jax.ShapeDtypeStruct((BATCH, FIELDS, EMBED), jnp.float32),
    scratch_types=[
        pltpu.VMEM((BLK, FIELDS), jnp.int32),
        pltpu.VMEM((CHUNK,), jnp.int32),
        pltpu.VMEM((CHUNK, EMBED), jnp.float32),
        pltpu.SemaphoreType.DMA,
        pltpu.SemaphoreType.DMA,
    ],
    compiler_params=pltpu.CompilerParams(use_tc_tiling_on_sc=False),
)
def _gather(idx_hbm, table_hbm, out_hbm, blk_v, idx_v, rows_v, gsem, ssem):
    wid = lax.axis_index("s") * NC + lax.axis_index("c")
    base = wid * ROWS_W

    def _flatten_step(r, carry):
        idx_v[pl.ds(r * FIELDS, LANES)] = blk_v[r, pl.ds(0, LANES)]
        idx_v[pl.ds(r * FIELDS + FIELDS - LANES, LANES)] = blk_v[
            r, pl.ds(FIELDS - LANES, LANES)
        ]
        return carry

    def _store_fire(k, r0):
        pltpu.async_copy(
            rows_v.at[pl.ds(k * FIELDS, FIELDS)], out_hbm.at[r0 + k], ssem
        )
        return r0

    def _store_drain(k, r0):
        pltpu.make_async_copy(
            rows_v.at[pl.ds(k * FIELDS, FIELDS)], out_hbm.at[r0 + k], ssem
        ).wait()
        return r0

    for c in range(NBLK):
        r0 = base + c * BLK
        pltpu.sync_copy(idx_hbm.at[pl.ds(r0, BLK)], blk_v)
        lax.fori_loop(0, BLK, _flatten_step, 0)
        pltpu.async_copy(table_hbm.at[idx_v], rows_v, gsem).wait()
        lax.fori_loop(0, BLK, _store_fire, r0)
        lax.fori_loop(0, BLK, _store_drain, r0)


def kernel(x, weight):
    return _gather(x, weight)
